# baseline (speedup 1.0000x reference)
import functools

import jax
import jax.numpy as jnp
from jax import lax
from jax.experimental import pallas as pl
from jax.experimental.pallas import tpu as pltpu
from jax.experimental.pallas import tpu_sc as plsc

B = 16384
D = 64
NREL = 5
NC = 2
NS = 16
NW = NC * NS
RPW = B // NW
CHUNK = 128
NCHUNK = RPW // CHUNK

_mesh = plsc.VectorSubcoreMesh(core_axis_name="c", subcore_axis_name="s")

_OUT = tuple(jax.ShapeDtypeStruct((B, D), jnp.float32) for _ in range(11))


@functools.partial(
    pl.kernel,
    mesh=_mesh,
    out_type=_OUT,
    scratch_types=[
        pltpu.VMEM((6 * NCHUNK, CHUNK), jnp.int32),
        pltpu.VMEM((RPW, D), jnp.float32),
        pltpu.VMEM((NREL, D), jnp.float32),
        pltpu.VMEM((NREL, D), jnp.float32),
        pltpu.VMEM((16,), jnp.float32),
        pltpu.SemaphoreType.DMA,
    ],
    compiler_params=pltpu.CompilerParams(needs_layout_passes=False,
                                         use_tc_tiling_on_sc=False),
)
def _sc_kernel(user_id, wrote, cited, coauthor, venue, affiliation,
               user_table, venue_table, affiliation_table, doc_embedding,
               relation_table, w_norm,
               out_user, out_wrote, out_cited, out_coauthor, out_venue,
               out_aff, out_r0, out_r1, out_r2, out_r3, out_r4,
               idx_v, rows_v, w_v, rel_v, s_buf, sem):
    wid = lax.axis_index("s") * NC + lax.axis_index("c")
    base = wid * RPW

    pltpu.sync_copy(w_norm, w_v)
    pltpu.sync_copy(relation_table, rel_v)

    lane = lax.iota(jnp.int32, 16)
    perms = [lane ^ k for k in (1, 2, 4, 8)]

    tasks = (
        (user_id, user_table, out_user, None),
        (wrote, doc_embedding, out_wrote, 0),
        (cited, doc_embedding, out_cited, 1),
        (coauthor, user_table, out_coauthor, 2),
        (venue, venue_table, out_venue, 3),
        (affiliation, affiliation_table, out_aff, 4),
    )

    for t, (ind_hbm, tab_hbm, out_hbm, r) in enumerate(tasks):
        for j in range(NCHUNK):
            pltpu.sync_copy(ind_hbm.at[pl.ds(base + j * CHUNK, CHUNK)],
                            idx_v.at[t * NCHUNK + j])
        copies = [
            pltpu.async_copy(tab_hbm.at[idx_v.at[t * NCHUNK + j]],
                             rows_v.at[pl.ds(j * CHUNK, CHUNK)], sem)
            for j in range(NCHUNK)
        ]
        for c in copies:
            c.wait()

        if r is not None:
            w0 = w_v[r, pl.ds(0, 16)]
            w1 = w_v[r, pl.ds(16, 16)]
            w2 = w_v[r, pl.ds(32, 16)]
            w3 = w_v[r, pl.ds(48, 16)]

            def row_body(i, carry, w0=w0, w1=w1, w2=w2, w3=w3):
                e0 = rows_v[i, pl.ds(0, 16)]
                e1 = rows_v[i, pl.ds(16, 16)]
                e2 = rows_v[i, pl.ds(32, 16)]
                e3 = rows_v[i, pl.ds(48, 16)]
                s = e0 * w0 + e1 * w1 + e2 * w2 + e3 * w3
                for perm in perms:
                    s_buf[...] = s
                    s = s + plsc.load_gather(s_buf, [perm])
                rows_v[i, pl.ds(0, 16)] = e0 - s * w0
                rows_v[i, pl.ds(16, 16)] = e1 - s * w1
                rows_v[i, pl.ds(32, 16)] = e2 - s * w2
                rows_v[i, pl.ds(48, 16)] = e3 - s * w3
                return carry

            lax.fori_loop(0, RPW, row_body, 0)

        pltpu.sync_copy(rows_v, out_hbm.at[pl.ds(base, RPW)])

    for r, out_hbm in enumerate((out_r0, out_r1, out_r2, out_r3, out_r4)):
        r0 = rel_v[r, pl.ds(0, 16)]
        r1 = rel_v[r, pl.ds(16, 16)]
        r2 = rel_v[r, pl.ds(32, 16)]
        r3 = rel_v[r, pl.ds(48, 16)]

        def fill_body(j, carry, r0=r0, r1=r1, r2=r2, r3=r3):
            rows_v[j, pl.ds(0, 16)] = r0
            rows_v[j, pl.ds(16, 16)] = r1
            rows_v[j, pl.ds(32, 16)] = r2
            rows_v[j, pl.ds(48, 16)] = r3
            return carry

        lax.fori_loop(0, CHUNK, fill_body, 0)
        for j in range(NCHUNK):
            pltpu.sync_copy(rows_v.at[pl.ds(0, CHUNK)],
                            out_hbm.at[pl.ds(base + j * CHUNK, CHUNK)])


def kernel(user_id, wrote, cited, coauthor, venue, affiliation,
           user_table, venue_table, affiliation_table, doc_embedding,
           relation_table, hyper_plane):
    nrm = jnp.sqrt(jnp.sum(hyper_plane * hyper_plane, axis=-1, keepdims=True))
    w_norm = hyper_plane / jnp.maximum(nrm, 1e-12)
    return _sc_kernel(user_id, wrote, cited, coauthor, venue, affiliation,
                      user_table, venue_table, affiliation_table,
                      doc_embedding, relation_table, w_norm)


# speedup vs baseline: 1.0599x; 1.0599x over previous
import functools

import jax
import jax.numpy as jnp
from jax import lax
from jax.experimental import pallas as pl
from jax.experimental.pallas import tpu as pltpu
from jax.experimental.pallas import tpu_sc as plsc

B = 16384
D = 64
NREL = 5
NC = 2
NS = 16
NW = NC * NS
RPW = B // NW
CHUNK = 128
NCHUNK = RPW // CHUNK

_mesh = plsc.VectorSubcoreMesh(core_axis_name="c", subcore_axis_name="s")

_OUT = tuple(jax.ShapeDtypeStruct((B, D), jnp.float32) for _ in range(11))


@functools.partial(
    pl.kernel,
    mesh=_mesh,
    out_type=_OUT,
    scratch_types=[
        pltpu.VMEM((6 * NCHUNK, CHUNK), jnp.int32),
        pltpu.VMEM((RPW, D), jnp.float32),
        pltpu.VMEM((NREL, D), jnp.float32),
        pltpu.VMEM((NREL, D), jnp.float32),
        pltpu.VMEM((8, 16), jnp.float32),
        pltpu.SemaphoreType.DMA,
    ],
    compiler_params=pltpu.CompilerParams(needs_layout_passes=False,
                                         use_tc_tiling_on_sc=False),
)
def _sc_kernel(user_id, wrote, cited, coauthor, venue, affiliation,
               user_table, venue_table, affiliation_table, doc_embedding,
               relation_table, w_norm,
               out_user, out_wrote, out_cited, out_coauthor, out_venue,
               out_aff, out_r0, out_r1, out_r2, out_r3, out_r4,
               idx_v, rows_v, w_v, rel_v, s_buf, sem):
    wid = lax.axis_index("s") * NC + lax.axis_index("c")
    base = wid * RPW

    pltpu.sync_copy(w_norm, w_v)
    pltpu.sync_copy(relation_table, rel_v)

    lane = lax.iota(jnp.int32, 16)
    perms = [lane ^ k for k in (1, 2, 4, 8)]

    tasks = (
        (user_id, user_table, out_user, None),
        (wrote, doc_embedding, out_wrote, 0),
        (cited, doc_embedding, out_cited, 1),
        (coauthor, user_table, out_coauthor, 2),
        (venue, venue_table, out_venue, 3),
        (affiliation, affiliation_table, out_aff, 4),
    )

    for t, (ind_hbm, tab_hbm, out_hbm, r) in enumerate(tasks):
        for j in range(NCHUNK):
            pltpu.sync_copy(ind_hbm.at[pl.ds(base + j * CHUNK, CHUNK)],
                            idx_v.at[t * NCHUNK + j])
        copies = [
            pltpu.async_copy(tab_hbm.at[idx_v.at[t * NCHUNK + j]],
                             rows_v.at[pl.ds(j * CHUNK, CHUNK)], sem)
            for j in range(NCHUNK)
        ]
        for c in copies:
            c.wait()

        if r is not None:
            w0 = w_v[r, pl.ds(0, 16)]
            w1 = w_v[r, pl.ds(16, 16)]
            w2 = w_v[r, pl.ds(32, 16)]
            w3 = w_v[r, pl.ds(48, 16)]

            U = 8

            def row_body(it, carry, w0=w0, w1=w1, w2=w2, w3=w3):
                i0 = it * U
                es = []
                ss = []
                for u in range(U):
                    e0 = rows_v[i0 + u, pl.ds(0, 16)]
                    e1 = rows_v[i0 + u, pl.ds(16, 16)]
                    e2 = rows_v[i0 + u, pl.ds(32, 16)]
                    e3 = rows_v[i0 + u, pl.ds(48, 16)]
                    es.append((e0, e1, e2, e3))
                    ss.append(e0 * w0 + e1 * w1 + e2 * w2 + e3 * w3)
                for perm in perms:
                    for u in range(U):
                        s_buf[u, pl.ds(0, 16)] = ss[u]
                    for u in range(U):
                        ss[u] = ss[u] + plsc.load_gather(s_buf.at[u], [perm])
                for u in range(U):
                    e0, e1, e2, e3 = es[u]
                    p = ss[u]
                    rows_v[i0 + u, pl.ds(0, 16)] = e0 - p * w0
                    rows_v[i0 + u, pl.ds(16, 16)] = e1 - p * w1
                    rows_v[i0 + u, pl.ds(32, 16)] = e2 - p * w2
                    rows_v[i0 + u, pl.ds(48, 16)] = e3 - p * w3
                return carry

            lax.fori_loop(0, RPW // U, row_body, 0)

        pltpu.sync_copy(rows_v, out_hbm.at[pl.ds(base, RPW)])

    for r, out_hbm in enumerate((out_r0, out_r1, out_r2, out_r3, out_r4)):
        r0 = rel_v[r, pl.ds(0, 16)]
        r1 = rel_v[r, pl.ds(16, 16)]
        r2 = rel_v[r, pl.ds(32, 16)]
        r3 = rel_v[r, pl.ds(48, 16)]

        def fill_body(j, carry, r0=r0, r1=r1, r2=r2, r3=r3):
            rows_v[j, pl.ds(0, 16)] = r0
            rows_v[j, pl.ds(16, 16)] = r1
            rows_v[j, pl.ds(32, 16)] = r2
            rows_v[j, pl.ds(48, 16)] = r3
            return carry

        lax.fori_loop(0, CHUNK, fill_body, 0)
        for j in range(NCHUNK):
            pltpu.sync_copy(rows_v.at[pl.ds(0, CHUNK)],
                            out_hbm.at[pl.ds(base + j * CHUNK, CHUNK)])


def kernel(user_id, wrote, cited, coauthor, venue, affiliation,
           user_table, venue_table, affiliation_table, doc_embedding,
           relation_table, hyper_plane):
    nrm = jnp.sqrt(jnp.sum(hyper_plane * hyper_plane, axis=-1, keepdims=True))
    w_norm = hyper_plane / jnp.maximum(nrm, 1e-12)
    return _sc_kernel(user_id, wrote, cited, coauthor, venue, affiliation,
                      user_table, venue_table, affiliation_table,
                      doc_embedding, relation_table, w_norm)


# speedup vs baseline: 1.1193x; 1.0560x over previous
import functools

import jax
import jax.numpy as jnp
from jax import lax
from jax.experimental import pallas as pl
from jax.experimental.pallas import tpu as pltpu
from jax.experimental.pallas import tpu_sc as plsc

B = 16384
D = 64
NREL = 5
NC = 2
NS = 16
NW = NC * NS
RPW = B // NW
CHUNK = 128
NCHUNK = RPW // CHUNK
U = 8

_mesh = plsc.VectorSubcoreMesh(core_axis_name="c", subcore_axis_name="s")

_params = pltpu.CompilerParams(needs_layout_passes=False,
                               use_tc_tiling_on_sc=False)


def _stage_indices(ind_hbms, base, idx_v, sem):
    copies = []
    for t, ind_hbm in enumerate(ind_hbms):
        for j in range(NCHUNK):
            copies.append(pltpu.async_copy(
                ind_hbm.at[pl.ds(base + j * CHUNK, CHUNK)],
                idx_v.at[t * NCHUNK + j], sem))
    return copies


def _gather(tab_hbm, t, idx_v, rows_v, sem):
    return [
        pltpu.async_copy(tab_hbm.at[idx_v.at[t * NCHUNK + j]],
                         rows_v.at[pl.ds(j * CHUNK, CHUNK)], sem)
        for j in range(NCHUNK)
    ]


def _project(rows_v, w_v, r, s_buf, perms):
    w0 = w_v[r, pl.ds(0, 16)]
    w1 = w_v[r, pl.ds(16, 16)]
    w2 = w_v[r, pl.ds(32, 16)]
    w3 = w_v[r, pl.ds(48, 16)]

    def row_body(it, carry):
        i0 = it * U
        es = []
        ss = []
        for u in range(U):
            e0 = rows_v[i0 + u, pl.ds(0, 16)]
            e1 = rows_v[i0 + u, pl.ds(16, 16)]
            e2 = rows_v[i0 + u, pl.ds(32, 16)]
            e3 = rows_v[i0 + u, pl.ds(48, 16)]
            es.append((e0, e1, e2, e3))
            ss.append(e0 * w0 + e1 * w1 + e2 * w2 + e3 * w3)
        for perm in perms:
            for u in range(U):
                s_buf[u, pl.ds(0, 16)] = ss[u]
            for u in range(U):
                ss[u] = ss[u] + plsc.load_gather(s_buf.at[u], [perm])
        for u in range(U):
            e0, e1, e2, e3 = es[u]
            p = ss[u]
            rows_v[i0 + u, pl.ds(0, 16)] = e0 - p * w0
            rows_v[i0 + u, pl.ds(16, 16)] = e1 - p * w1
            rows_v[i0 + u, pl.ds(32, 16)] = e2 - p * w2
            rows_v[i0 + u, pl.ds(48, 16)] = e3 - p * w3
        return carry

    lax.fori_loop(0, RPW // U, row_body, 0)


@functools.partial(
    pl.kernel,
    mesh=_mesh,
    out_type=tuple(jax.ShapeDtypeStruct((B, D), jnp.float32)
                   for _ in range(9)),
    scratch_types=[
        pltpu.VMEM((4 * NCHUNK, CHUNK), jnp.int32),
        pltpu.VMEM((RPW, D), jnp.float32),
        pltpu.VMEM((NREL, D), jnp.float32),
        pltpu.VMEM((NREL, D), jnp.float32),
        pltpu.VMEM((U, 16), jnp.float32),
        pltpu.SemaphoreType.DMA,
    ],
    compiler_params=_params,
)
def _sc_kernel_a(user_id, coauthor, venue, affiliation,
                 user_table, venue_table, affiliation_table,
                 relation_table, w_norm,
                 out_user, out_coauthor, out_venue, out_aff,
                 out_r0, out_r1, out_r2, out_r3, out_r4,
                 idx_v, rows_v, w_v, rel_v, s_buf, sem):
    wid = lax.axis_index("s") * NC + lax.axis_index("c")
    base = wid * RPW

    pltpu.sync_copy(w_norm, w_v)
    pltpu.sync_copy(relation_table, rel_v)

    lane = lax.iota(jnp.int32, 16)
    perms = [lane ^ k for k in (1, 2, 4, 8)]

    tasks = (
        (user_table, out_user, None),
        (user_table, out_coauthor, 2),
        (venue_table, out_venue, 3),
        (affiliation_table, out_aff, 4),
    )
    for c in _stage_indices((user_id, coauthor, venue, affiliation),
                            base, idx_v, sem):
        c.wait()

    for t, (tab_hbm, out_hbm, r) in enumerate(tasks):
        for c in _gather(tab_hbm, t, idx_v, rows_v, sem):
            c.wait()
        if r is not None:
            _project(rows_v, w_v, r, s_buf, perms)
        pltpu.sync_copy(rows_v, out_hbm.at[pl.ds(base, RPW)])

    for r, out_hbm in enumerate((out_r0, out_r1, out_r2, out_r3, out_r4)):
        r0 = rel_v[r, pl.ds(0, 16)]
        r1 = rel_v[r, pl.ds(16, 16)]
        r2 = rel_v[r, pl.ds(32, 16)]
        r3 = rel_v[r, pl.ds(48, 16)]

        def fill_body(j, carry, r0=r0, r1=r1, r2=r2, r3=r3):
            rows_v[j, pl.ds(0, 16)] = r0
            rows_v[j, pl.ds(16, 16)] = r1
            rows_v[j, pl.ds(32, 16)] = r2
            rows_v[j, pl.ds(48, 16)] = r3
            return carry

        lax.fori_loop(0, CHUNK, fill_body, 0)
        for j in range(NCHUNK):
            pltpu.sync_copy(rows_v.at[pl.ds(0, CHUNK)],
                            out_hbm.at[pl.ds(base + j * CHUNK, CHUNK)])


@functools.partial(
    pl.kernel,
    mesh=_mesh,
    out_type=tuple(jax.ShapeDtypeStruct((B, D), jnp.float32)
                   for _ in range(2)),
    scratch_types=[
        pltpu.VMEM((2 * NCHUNK, CHUNK), jnp.int32),
        pltpu.VMEM((2, RPW, D), jnp.float32),
        pltpu.VMEM((NREL, D), jnp.float32),
        pltpu.VMEM((U, 16), jnp.float32),
        pltpu.SemaphoreType.DMA,
        pltpu.SemaphoreType.DMA,
    ],
    compiler_params=_params,
)
def _sc_kernel_b(wrote, cited, doc_embedding, w_norm,
                 out_wrote, out_cited,
                 idx_v, rows2_v, w_v, s_buf, sem0, sem1):
    wid = lax.axis_index("s") * NC + lax.axis_index("c")
    base = wid * RPW

    pltpu.sync_copy(w_norm, w_v)

    lane = lax.iota(jnp.int32, 16)
    perms = [lane ^ k for k in (1, 2, 4, 8)]

    for c in _stage_indices((wrote, cited), base, idx_v, sem0):
        c.wait()

    g0 = _gather(doc_embedding, 0, idx_v, rows2_v.at[0], sem0)
    g1 = _gather(doc_embedding, 1, idx_v, rows2_v.at[1], sem1)
    for t, (out_hbm, r, g) in enumerate(((out_wrote, 0, g0),
                                         (out_cited, 1, g1))):
        for c in g:
            c.wait()
        _project(rows2_v.at[t], w_v, r, s_buf, perms)
        pltpu.sync_copy(rows2_v.at[t], out_hbm.at[pl.ds(base, RPW)])


def kernel(user_id, wrote, cited, coauthor, venue, affiliation,
           user_table, venue_table, affiliation_table, doc_embedding,
           relation_table, hyper_plane):
    nrm = jnp.sqrt(jnp.sum(hyper_plane * hyper_plane, axis=-1, keepdims=True))
    w_norm = hyper_plane / jnp.maximum(nrm, 1e-12)
    (user_embs, coauthor_embs, venue_embs, affiliation_embs,
     wrote_rel, cited_rel, co_author_rel, venue_rel, affiliation_rel) = (
        _sc_kernel_a(user_id, coauthor, venue, affiliation,
                     user_table, venue_table, affiliation_table,
                     relation_table, w_norm))
    wrote_embs, cited_embs = _sc_kernel_b(wrote, cited, doc_embedding,
                                          w_norm)
    return (user_embs, wrote_embs, cited_embs, coauthor_embs, venue_embs,
            affiliation_embs, wrote_rel, cited_rel, co_author_rel,
            venue_rel, affiliation_rel)


# speedup vs baseline: 1.2148x; 1.0853x over previous
import functools

import jax
import jax.numpy as jnp
from jax import lax
from jax.experimental import pallas as pl
from jax.experimental.pallas import tpu as pltpu
from jax.experimental.pallas import tpu_sc as plsc

B = 16384
D = 64
NREL = 5
NC = 2
NS = 16
NW = NC * NS
RPW = B // NW
CHUNK = 128
NCHUNK = RPW // CHUNK
U = 8

_mesh = plsc.VectorSubcoreMesh(core_axis_name="c", subcore_axis_name="s")

_params = pltpu.CompilerParams(needs_layout_passes=False,
                               use_tc_tiling_on_sc=False)


def _stage_indices(ind_hbms, base, idx_v, sem):
    copies = []
    for t, ind_hbm in enumerate(ind_hbms):
        for j in range(NCHUNK):
            copies.append(pltpu.async_copy(
                ind_hbm.at[pl.ds(base + j * CHUNK, CHUNK)],
                idx_v.at[t * NCHUNK + j], sem))
    return copies


def _gather(tab_hbm, t, idx_v, rows_v, sem):
    return [
        pltpu.async_copy(tab_hbm.at[idx_v.at[t * NCHUNK + j]],
                         rows_v.at[pl.ds(j * CHUNK, CHUNK)], sem)
        for j in range(NCHUNK)
    ]


def _project(rows_v, w_v, r, s_buf, perms):
    w0 = w_v[r, pl.ds(0, 16)]
    w1 = w_v[r, pl.ds(16, 16)]
    w2 = w_v[r, pl.ds(32, 16)]
    w3 = w_v[r, pl.ds(48, 16)]

    def row_body(it, carry):
        i0 = it * U
        es = []
        ss = []
        for u in range(U):
            e0 = rows_v[i0 + u, pl.ds(0, 16)]
            e1 = rows_v[i0 + u, pl.ds(16, 16)]
            e2 = rows_v[i0 + u, pl.ds(32, 16)]
            e3 = rows_v[i0 + u, pl.ds(48, 16)]
            es.append((e0, e1, e2, e3))
            ss.append(e0 * w0 + e1 * w1 + e2 * w2 + e3 * w3)
        for perm in perms:
            for u in range(U):
                s_buf[u, pl.ds(0, 16)] = ss[u]
            for u in range(U):
                ss[u] = ss[u] + plsc.load_gather(s_buf.at[u], [perm])
        for u in range(U):
            e0, e1, e2, e3 = es[u]
            p = ss[u]
            rows_v[i0 + u, pl.ds(0, 16)] = e0 - p * w0
            rows_v[i0 + u, pl.ds(16, 16)] = e1 - p * w1
            rows_v[i0 + u, pl.ds(32, 16)] = e2 - p * w2
            rows_v[i0 + u, pl.ds(48, 16)] = e3 - p * w3
        return carry

    lax.fori_loop(0, RPW // U, row_body, 0)


@functools.partial(
    pl.kernel,
    mesh=_mesh,
    out_type=tuple(jax.ShapeDtypeStruct((B, D), jnp.float32)
                   for _ in range(9)),
    scratch_types=[
        pltpu.VMEM((4 * NCHUNK, CHUNK), jnp.int32),
        pltpu.VMEM((RPW, D), jnp.float32),
        pltpu.VMEM((NREL, D), jnp.float32),
        pltpu.VMEM((NREL, D), jnp.float32),
        pltpu.VMEM((U, 16), jnp.float32),
        pltpu.SemaphoreType.DMA,
    ],
    compiler_params=_params,
)
def _sc_kernel_a(user_id, coauthor, venue, affiliation,
                 user_table, venue_table, affiliation_table,
                 relation_table, w_norm,
                 out_user, out_coauthor, out_venue, out_aff,
                 out_r0, out_r1, out_r2, out_r3, out_r4,
                 idx_v, rows_v, w_v, rel_v, s_buf, sem):
    wid = lax.axis_index("s") * NC + lax.axis_index("c")
    base = wid * RPW

    pltpu.sync_copy(w_norm, w_v)
    pltpu.sync_copy(relation_table, rel_v)

    lane = lax.iota(jnp.int32, 16)
    perms = [lane ^ k for k in (1, 2, 4, 8)]

    tasks = (
        (user_table, out_user, None),
        (user_table, out_coauthor, 2),
        (venue_table, out_venue, 3),
        (affiliation_table, out_aff, 4),
    )
    for c in _stage_indices((user_id, coauthor, venue, affiliation),
                            base, idx_v, sem):
        c.wait()

    for t, (tab_hbm, out_hbm, r) in enumerate(tasks):
        for c in _gather(tab_hbm, t, idx_v, rows_v, sem):
            c.wait()
        if r is not None:
            _project(rows_v, w_v, r, s_buf, perms)
        pltpu.sync_copy(rows_v, out_hbm.at[pl.ds(base, RPW)])

    for r, out_hbm in enumerate((out_r0, out_r1, out_r2, out_r3, out_r4)):
        r0 = rel_v[r, pl.ds(0, 16)]
        r1 = rel_v[r, pl.ds(16, 16)]
        r2 = rel_v[r, pl.ds(32, 16)]
        r3 = rel_v[r, pl.ds(48, 16)]

        def fill_body(j, carry, r0=r0, r1=r1, r2=r2, r3=r3):
            rows_v[j, pl.ds(0, 16)] = r0
            rows_v[j, pl.ds(16, 16)] = r1
            rows_v[j, pl.ds(32, 16)] = r2
            rows_v[j, pl.ds(48, 16)] = r3
            return carry

        lax.fori_loop(0, CHUNK, fill_body, 0)
        for j in range(NCHUNK):
            pltpu.sync_copy(rows_v.at[pl.ds(0, CHUNK)],
                            out_hbm.at[pl.ds(base + j * CHUNK, CHUNK)])


CH = CHUNK
NCH = RPW // CH
UB = 8


@functools.partial(
    pl.kernel,
    mesh=_mesh,
    out_type=tuple(jax.ShapeDtypeStruct((B, 2 * D), jnp.float32)
                   for _ in range(2)),
    scratch_types=[
        pltpu.VMEM((2 * NCHUNK, CHUNK), jnp.int32),
        pltpu.VMEM((2, CH, 2 * D), jnp.float32),
        pltpu.VMEM((2, CH, 2 * D), jnp.float32),
        pltpu.VMEM((NREL, D), jnp.float32),
        pltpu.VMEM((UB, 16), jnp.float32),
        pltpu.SemaphoreType.DMA,
        pltpu.SemaphoreType.DMA,
        pltpu.SemaphoreType.DMA,
        pltpu.SemaphoreType.DMA,
        pltpu.SemaphoreType.DMA,
    ],
    compiler_params=pltpu.CompilerParams(needs_layout_passes=False,
                                         use_tc_tiling_on_sc=True),
)
def _sc_kernel_b(wrote, cited, doc_pad, w_norm,
                 out_wrote, out_cited,
                 idx_v, gbuf, rbuf, w_v, s_buf,
                 isem, gsem0, gsem1, osem0, osem1):
    gsems = (gsem0, gsem1)
    osems = (osem0, osem1)
    wid = lax.axis_index("s") * NC + lax.axis_index("c")
    base = wid * RPW

    pltpu.sync_copy(w_norm, w_v)

    lane = lax.iota(jnp.int32, 16)
    perms = [lane ^ k for k in (1, 2, 4, 8)]

    for c in _stage_indices((wrote, cited), base, idx_v, isem):
        c.wait()

    def fire(k):
        return pltpu.async_copy(doc_pad.at[idx_v.at[k]], gbuf.at[k % 2],
                                gsems[k % 2])

    g = {0: fire(0)}
    o = {}
    ws = [[w_v[r, pl.ds(c * 16, 16)] for c in range(4)] for r in range(2)]

    for k in range(2 * NCH):
        p = k % 2
        t, kk = divmod(k, NCH)
        if k + 1 < 2 * NCH:
            g[k + 1] = fire(k + 1)
        g.pop(k).wait()
        if k - 2 in o:
            o.pop(k - 2).wait()
        wr = ws[t]

        def ext_body(it, carry, p=p, wr=wr):
            for u in range(UB):
                row = it * UB + u
                es = [gbuf[p, row, pl.ds(c * 16, 16)] for c in range(4)]
                s = (es[0] * wr[0] + es[1] * wr[1]
                     + es[2] * wr[2] + es[3] * wr[3])
                for perm in perms:
                    s_buf[u, pl.ds(0, 16)] = s
                    s = s + plsc.load_gather(s_buf.at[u], [perm])
                for c in range(4):
                    rbuf[p, row, pl.ds(c * 16, 16)] = es[c] - s * wr[c]
            return carry

        lax.fori_loop(0, CH // UB, ext_body, 0)
        out_hbm = out_wrote if t == 0 else out_cited
        o[k] = pltpu.async_copy(rbuf.at[p],
                                out_hbm.at[pl.ds(base + kk * CH, CH)],
                                osems[p])
    for c in o.values():
        c.wait()


def kernel(user_id, wrote, cited, coauthor, venue, affiliation,
           user_table, venue_table, affiliation_table, doc_embedding,
           relation_table, hyper_plane):
    nrm = jnp.sqrt(jnp.sum(hyper_plane * hyper_plane, axis=-1, keepdims=True))
    w_norm = hyper_plane / jnp.maximum(nrm, 1e-12)
    (user_embs, coauthor_embs, venue_embs, affiliation_embs,
     wrote_rel, cited_rel, co_author_rel, venue_rel, affiliation_rel) = (
        _sc_kernel_a(user_id, coauthor, venue, affiliation,
                     user_table, venue_table, affiliation_table,
                     relation_table, w_norm))
    doc_pad = jnp.pad(doc_embedding, ((0, 0), (0, D)))
    wrote_pad, cited_pad = _sc_kernel_b(wrote, cited, doc_pad, w_norm)
    wrote_embs = wrote_pad[:, :D]
    cited_embs = cited_pad[:, :D]
    return (user_embs, wrote_embs, cited_embs, coauthor_embs, venue_embs,
            affiliation_embs, wrote_rel, cited_rel, co_author_rel,
            venue_rel, affiliation_rel)
